# baseline (speedup 1.0000x reference)
import functools

import jax
import jax.numpy as jnp
from jax import lax
from jax.experimental import pallas as pl
from jax.experimental.pallas import tpu as pltpu
from jax.experimental.pallas import tpu_sc as plsc

BATCH = 16384
NUM_NUMERIC = 13
N_CAT = 26
VOCAB = 100000
EMB_DIM = 32
TOTAL_ROWS = BATCH * N_CAT

_INFO = plsc.get_sparse_core_info()
NC = _INFO.num_cores
NS = _INFO.num_subcores
NW = NC * NS
ROWS_W = TOTAL_ROWS // NW
IGRP = 128
NGRP = ROWS_W // IGRP
GPC = 13
NCHUNK = NGRP // GPC
CROWS = GPC * IGRP

_mesh = plsc.VectorSubcoreMesh(core_axis_name="c", subcore_axis_name="s")


@functools.partial(
    pl.kernel,
    mesh=_mesh,
    compiler_params=pltpu.CompilerParams(use_tc_tiling_on_sc=False),
    out_type=jax.ShapeDtypeStruct((TOTAL_ROWS, EMB_DIM), jnp.float32),
    scratch_types=[
        pltpu.VMEM((NGRP, IGRP), jnp.int32),
        pltpu.VMEM((CROWS, EMB_DIM), jnp.float32),
        pltpu.SemaphoreType.DMA,
    ],
)
def _gather_rows(idx_hbm, table_hbm, out_hbm, idx_v, rows_v, sem):
    wid = lax.axis_index("s") * NC + lax.axis_index("c")
    pltpu.sync_copy(idx_hbm.at[wid], idx_v)
    for c in range(NCHUNK):
        copies = [
            pltpu.async_copy(
                table_hbm.at[idx_v.at[c * GPC + g]],
                rows_v.at[pl.ds(g * IGRP, IGRP)],
                sem,
            )
            for g in range(GPC)
        ]
        for cp in copies:
            cp.wait()
        pltpu.sync_copy(
            rows_v,
            out_hbm.at[pl.ds(wid * ROWS_W + c * CROWS, CROWS)],
        )


def kernel(numeric, categorical, tables):
    idx = categorical.astype(jnp.int32) + jnp.arange(
        N_CAT, dtype=jnp.int32
    ) * VOCAB
    idx = idx.reshape(NW, NGRP, IGRP)
    table_flat = tables.reshape(N_CAT * VOCAB, EMB_DIM)
    emb = _gather_rows(idx, table_flat)
    return jnp.concatenate([numeric, emb.reshape(BATCH, N_CAT * EMB_DIM)], axis=1)
